# baseline (speedup 1.0000x reference)
import functools

import jax
import jax.numpy as jnp
from jax import lax
from jax.experimental import pallas as pl
from jax.experimental.pallas import tpu as pltpu
from jax.experimental.pallas import tpu_sc as plsc

DIM = 1024
E = 8
K = 2
HID = 2048

TM = 128
RT = 512

INTERPRET = False


def _router_body(x_ref, wg_ref, val_ref, idx_ref):
    x = x_ref[...]
    logits = lax.dot_general(x, wg_ref[...], (((1,), (1,)), ((), ())),
                             preferred_element_type=jnp.float32)
    m = jnp.max(logits, axis=-1, keepdims=True)
    ex = jnp.exp(logits - m)
    gates = ex / jnp.sum(ex, axis=-1, keepdims=True)
    ii = lax.broadcasted_iota(jnp.int32, gates.shape, 1)
    v1 = jnp.max(gates, axis=-1, keepdims=True)
    i1 = jnp.min(jnp.where(gates == v1, ii, E), axis=-1, keepdims=True)
    g2 = jnp.where(ii == i1, -1.0, gates)
    v2 = jnp.max(g2, axis=-1, keepdims=True)
    i2 = jnp.min(jnp.where(g2 == v2, ii, E), axis=-1, keepdims=True)
    val_ref[...] = jnp.concatenate([v1, v2], axis=1)
    idx_ref[...] = jnp.concatenate([i1, i2], axis=1)


def _router(x2, w_gate):
    n = x2.shape[0]
    return pl.pallas_call(
        _router_body,
        grid=(n // RT,),
        in_specs=[
            pl.BlockSpec((RT, DIM), lambda t: (t, 0)),
            pl.BlockSpec((E, DIM), lambda t: (0, 0)),
        ],
        out_specs=[
            pl.BlockSpec((RT, K), lambda t: (t, 0)),
            pl.BlockSpec((RT, K), lambda t: (t, 0)),
        ],
        out_shape=[
            jax.ShapeDtypeStruct((n, K), jnp.float32),
            jax.ShapeDtypeStruct((n, K), jnp.int32),
        ],
        interpret=INTERPRET,
    )(x2, w_gate)


def _ffn_body(meta_ref, xs_ref, wi_ref, wg_ref, wo_ref, sc_ref, ys_ref,
              wib, wgb, wob, sem0, sem1):
    t = pl.program_id(0)
    eid = meta_ref[0, t]
    rs = meta_ref[1, t]
    slot = meta_ref[2, t]
    peid = meta_ref[3, t]

    def fetch(e, s, ssem):
        for src, dst in ((wi_ref, wib), (wg_ref, wgb), (wo_ref, wob)):
            pltpu.make_async_copy(src.at[e], dst.at[s], ssem).start()

    def wait(e, s, ssem):
        for src, dst in ((wi_ref, wib), (wg_ref, wgb), (wo_ref, wob)):
            pltpu.make_async_copy(src.at[e], dst.at[s], ssem).wait()

    @pl.when(t == 0)
    def _():
        fetch(eid, 0, sem0)

    @pl.when(jnp.logical_and(rs == 1, peid >= 0))
    def _():
        @pl.when(slot == 0)
        def _():
            fetch(peid, 1, sem1)

        @pl.when(slot == 1)
        def _():
            fetch(peid, 0, sem0)

    @pl.when(rs == 1)
    def _():
        @pl.when(slot == 0)
        def _():
            wait(eid, 0, sem0)

        @pl.when(slot == 1)
        def _():
            wait(eid, 1, sem1)

    @pl.when(meta_ref[4, t] == 1)
    def _():
        cdt = jnp.bfloat16
        x = xs_ref[...].astype(cdt)
        up = lax.dot_general(x, wib[slot].astype(cdt), (((1,), (1,)), ((), ())),
                             preferred_element_type=jnp.float32)
        g = lax.dot_general(x, wgb[slot].astype(cdt), (((1,), (1,)), ((), ())),
                            preferred_element_type=jnp.float32)
        h = (g * jax.nn.sigmoid(g) * up).astype(cdt)
        y = lax.dot_general(h, wob[slot].astype(cdt), (((1,), (1,)), ((), ())),
                            preferred_element_type=jnp.float32)
        scale = sc_ref[0, 0, :].reshape(TM, 1)
        ys_ref[...] = y * scale


def _ffn(meta, xs, wi, wg, wo, scale3):
    s = xs.shape[0]
    nt = s // TM
    grid_spec = pltpu.PrefetchScalarGridSpec(
        num_scalar_prefetch=1,
        grid=(nt,),
        in_specs=[
            pl.BlockSpec((TM, DIM), lambda t, m: (t, 0)),
            pl.BlockSpec(memory_space=pltpu.MemorySpace.HBM),
            pl.BlockSpec(memory_space=pltpu.MemorySpace.HBM),
            pl.BlockSpec(memory_space=pltpu.MemorySpace.HBM),
            pl.BlockSpec((1, 1, TM), lambda t, m: (t, 0, 0)),
        ],
        out_specs=pl.BlockSpec((TM, DIM), lambda t, m: (t, 0)),
        scratch_shapes=[
            pltpu.VMEM((2, HID, DIM), jnp.float32),
            pltpu.VMEM((2, HID, DIM), jnp.float32),
            pltpu.VMEM((2, DIM, HID), jnp.float32),
            pltpu.SemaphoreType.DMA,
            pltpu.SemaphoreType.DMA,
        ],
    )
    return pl.pallas_call(
        _ffn_body,
        grid_spec=grid_spec,
        out_shape=jax.ShapeDtypeStruct((s, DIM), jnp.float32),
        interpret=INTERPRET,
    )(meta, xs, wi, wg, wo, scale3)


NP = 4096 * K
S = NP + E * TM
NSUB = 16
CH = NP // NSUB
NV = CH // 16
STRIPE = S // 32
GB = 64
CB = 32


RB = 32
NB = CH // RB


def _dispatch_body(idx_hbm, val_hbm, x_hbm,
                   xs_hbm, scale_hbm, pos_hbm, cnt_hbm,
                   e_v, v_v, posf_v, pos2_v, cnt_v, allcnt_v,
                   pos8_v, tok8_v, rows_v,
                   counts_sh, sem, sem2):
    cid = lax.axis_index("c")
    sid = lax.axis_index("s")
    lane = lax.broadcasted_iota(jnp.int32, (16,), 0)
    zero16 = jnp.zeros((16,), jnp.int32)
    pbase = sid * CH

    pltpu.sync_copy(idx_hbm.at[pl.ds(pbase, CH)], e_v)
    pltpu.sync_copy(val_hbm.at[pl.ds(pbase, CH)], v_v)

    def cnt_body(i, c):
        ev = e_v[pl.ds(i * 16, 16)]
        for e in range(E):
            incl = plsc.cumsum(jnp.where(ev == e, 1, 0))
            pc = jnp.broadcast_to(jnp.max(incl, axis=0), (16,))
            c = c + jnp.where(lane == e, pc, 0)
        return c
    cnt = lax.fori_loop(0, NV, cnt_body, zero16)
    cnt_v[...] = cnt
    pltpu.sync_copy(cnt_v, counts_sh.at[pl.ds(sid * 16, 16)])
    plsc.subcore_barrier()

    pltpu.sync_copy(counts_sh, allcnt_v)

    def acc_body(i, carry):
        tot, pre = carry
        row = allcnt_v[pl.ds(i * 16, 16)]
        return tot + row, pre + jnp.where(i < sid, row, 0)
    tot, pre = lax.fori_loop(0, NSUB, acc_body, (zero16, zero16))
    padded = ((tot + TM - 1) // TM) * TM
    starts = plsc.cumsum(padded) - padded
    base0 = starts + pre

    @pl.when(jnp.logical_and(cid == 0, sid == 0))
    def _():
        cnt_v[...] = tot
        pltpu.sync_copy(cnt_v, cnt_hbm)

    def pos_body(i, b):
        ev = e_v[pl.ds(i * 16, 16)]
        p = zero16
        for e in range(E):
            m = ev == e
            incl = plsc.cumsum(jnp.where(m, 1, 0))
            be = jnp.broadcast_to(jnp.max(jnp.where(lane == e, b, -1), axis=0),
                                  (16,))
            p = jnp.where(m, be + incl - 1, p)
            pc = jnp.broadcast_to(jnp.max(incl, axis=0), (16,))
            b = b + jnp.where(lane == e, pc, 0)
        posf_v[pl.ds(i * 16, 16)] = p
        pos2_v[i // 8, pl.ds((i % 8) * 16, 16)] = p
        tok = (pbase + i * 16 + lane) // K
        nl = RB // 16
        pos8_v[i // nl, pl.ds((i % nl) * 16, 16)] = p
        tok8_v[i // nl, pl.ds((i % nl) * 16, 16)] = tok
        return b
    lax.fori_loop(0, NV, pos_body, base0)

    @pl.when(cid == 0)
    def _():
        pltpu.sync_copy(posf_v, pos_hbm.at[pl.ds(pbase, CH)])

    co = pl.multiple_of(cid * (CH // 2), CH // 2)
    for j in range(CH // 256):
        jo = pl.multiple_of(co // 128 + j, 1)
        pltpu.sync_copy(v_v.at[pl.ds(co + j * 128, 128)],
                        scale_hbm.at[pos2_v.at[jo]])

    bo = cid * (NB // 2)
    gcp = pltpu.async_copy(x_hbm.at[tok8_v.at[bo]], rows_v.at[0], sem)
    scp = None
    for b in range(NB // 2):
        cur = b % 2
        if scp is not None:
            scp.wait()
        if b + 1 < NB // 2:
            ncp = pltpu.async_copy(x_hbm.at[tok8_v.at[bo + b + 1]],
                                   rows_v.at[1 - cur], sem)
        gcp.wait()
        scp = pltpu.async_copy(rows_v.at[cur], xs_hbm.at[pos8_v.at[bo + b]],
                               sem2)
        if b + 1 < NB // 2:
            gcp = ncp
    scp.wait()


def _dispatch(idx, vals, x2):
    n = x2.shape[0]
    mesh = plsc.VectorSubcoreMesh(core_axis_name="c", subcore_axis_name="s")
    f = pl.kernel(
        _dispatch_body,
        out_type=[
            jax.ShapeDtypeStruct((S, DIM), jnp.float32),
            jax.ShapeDtypeStruct((S,), jnp.float32),
            jax.ShapeDtypeStruct((NP,), jnp.int32),
            jax.ShapeDtypeStruct((16,), jnp.int32),
        ],
        mesh=mesh,
        scratch_types=[
            pltpu.VMEM((CH,), jnp.int32),
            pltpu.VMEM((CH,), jnp.float32),
            pltpu.VMEM((CH,), jnp.int32),
            pltpu.VMEM((CH // 128, 128), jnp.int32),
            pltpu.VMEM((16,), jnp.int32),
            pltpu.VMEM((NSUB * 16,), jnp.int32),
            pltpu.VMEM((NB, RB), jnp.int32),
            pltpu.VMEM((NB, RB), jnp.int32),
            pltpu.VMEM((2, RB, DIM), jnp.float32),
            pltpu.VMEM_SHARED((NSUB * 16,), jnp.int32),
            pltpu.SemaphoreType.DMA,
            pltpu.SemaphoreType.DMA,
        ],
        compiler_params=pltpu.CompilerParams(needs_layout_passes=False),
    )
    return f(idx.reshape(NP), vals.reshape(NP), x2)


def _combine_body(ys_hbm, pos_hbm, out_hbm, posc_v, rows_v, out_v, sem, sem2):
    cid = lax.axis_index("c")
    sid = lax.axis_index("s")
    wid = sid * 2 + cid
    rbase = pl.multiple_of(wid * (NP // 32), NP // 32)
    nb = (NP // 32) // CB
    for b in range(nb):
        pltpu.sync_copy(pos_hbm.at[pl.ds(rbase + b * CB, CB)], posc_v.at[b])
    gcp = pltpu.async_copy(ys_hbm.at[posc_v.at[0]], rows_v.at[0], sem)
    ocp = None
    for b in range(nb):
        cur = b % 2
        if b + 1 < nb:
            ncp = pltpu.async_copy(ys_hbm.at[posc_v.at[b + 1]],
                                   rows_v.at[1 - cur], sem)
        gcp.wait()
        if ocp is not None:
            ocp.wait()

        def add_body(j, _):
            for t in range(CB // K):
                a = rows_v[cur, K * t, pl.ds(j * 16, 16)]
                c = rows_v[cur, K * t + 1, pl.ds(j * 16, 16)]
                out_v[cur, t, pl.ds(j * 16, 16)] = a + c
            return 0
        lax.fori_loop(0, DIM // 16, add_body, 0)
        obase = pl.multiple_of((rbase + b * CB) // K, CB // K)
        ocp = pltpu.async_copy(out_v.at[cur], out_hbm.at[pl.ds(obase, CB // K)],
                               sem2)
        if b + 1 < nb:
            gcp = ncp
    ocp.wait()


def _combine(ys, pos, n):
    mesh = plsc.VectorSubcoreMesh(core_axis_name="c", subcore_axis_name="s")
    f = pl.kernel(
        _combine_body,
        out_type=jax.ShapeDtypeStruct((n, DIM), jnp.float32),
        mesh=mesh,
        scratch_types=[
            pltpu.VMEM(((NP // 32) // CB, CB), jnp.int32),
            pltpu.VMEM((2, CB, DIM), jnp.float32),
            pltpu.VMEM((2, CB // K, DIM), jnp.float32),
            pltpu.SemaphoreType.DMA,
            pltpu.SemaphoreType.DMA,
        ],
        compiler_params=pltpu.CompilerParams(needs_layout_passes=False),
    )
    return f(ys, pos)


def kernel(x, w_gate, wi, wg, wo):
    b, t, _ = x.shape
    n = b * t
    nt = S // TM

    x2 = x.reshape(n, DIM)
    vals, idx = _router(x2, w_gate)

    xs, scale, pos, cnt16 = _dispatch(idx, vals, x2)

    counts = cnt16[:E]
    padded = ((counts + TM - 1) // TM) * TM
    ends = jnp.cumsum(padded).astype(jnp.int32)
    idxs = jnp.arange(nt, dtype=jnp.int32)
    gid = jnp.minimum(
        jnp.sum((idxs[:, None] * TM) >= ends[None, :], axis=1),
        E - 1).astype(jnp.int32)
    rs = jnp.concatenate([jnp.ones((1,), jnp.int32),
                          (gid[1:] != gid[:-1]).astype(jnp.int32)])
    run_id = jnp.cumsum(rs).astype(jnp.int32) - 1
    slot = run_id % 2
    ns = jnp.where(rs == 1, idxs, 2 * nt)
    suf = lax.associative_scan(jnp.minimum, ns[::-1])[::-1]
    nxt = jnp.concatenate([suf[1:], jnp.full((1,), 2 * nt, jnp.int32)])
    peid = jnp.where((rs == 1) & (nxt < nt), gid[jnp.clip(nxt, 0, nt - 1)], -1)
    act = (idxs * TM < ends[E - 1]).astype(jnp.int32)
    meta = jnp.stack([gid, rs, slot, peid.astype(jnp.int32), act])

    ys = _ffn(meta, xs, wi, wg, wo, scale.reshape(nt, 1, TM))

    out = _combine(ys, pos, n)
    return out.reshape(b, t, DIM)


# speedup vs baseline: 1.5598x; 1.5598x over previous
import functools

import jax
import jax.numpy as jnp
from jax import lax
from jax.experimental import pallas as pl
from jax.experimental.pallas import tpu as pltpu
from jax.experimental.pallas import tpu_sc as plsc

DIM = 1024
E = 8
K = 2
HID = 2048

TM = 256
RT = 512

INTERPRET = False


def _router_body(x_ref, wg_ref, val_ref, idx_ref):
    x = x_ref[...]
    logits = lax.dot_general(x, wg_ref[...], (((1,), (1,)), ((), ())),
                             preferred_element_type=jnp.float32)
    m = jnp.max(logits, axis=-1, keepdims=True)
    ex = jnp.exp(logits - m)
    gates = ex / jnp.sum(ex, axis=-1, keepdims=True)
    ii = lax.broadcasted_iota(jnp.int32, gates.shape, 1)
    v1 = jnp.max(gates, axis=-1, keepdims=True)
    i1 = jnp.min(jnp.where(gates == v1, ii, E), axis=-1, keepdims=True)
    g2 = jnp.where(ii == i1, -1.0, gates)
    v2 = jnp.max(g2, axis=-1, keepdims=True)
    i2 = jnp.min(jnp.where(g2 == v2, ii, E), axis=-1, keepdims=True)
    val_ref[...] = jnp.concatenate([v1, v2], axis=1)
    idx_ref[...] = jnp.concatenate([i1, i2], axis=1)


def _router(x2, w_gate):
    n = x2.shape[0]
    return pl.pallas_call(
        _router_body,
        grid=(n // RT,),
        in_specs=[
            pl.BlockSpec((RT, DIM), lambda t: (t, 0)),
            pl.BlockSpec((E, DIM), lambda t: (0, 0)),
        ],
        out_specs=[
            pl.BlockSpec((RT, K), lambda t: (t, 0)),
            pl.BlockSpec((RT, K), lambda t: (t, 0)),
        ],
        out_shape=[
            jax.ShapeDtypeStruct((n, K), jnp.float32),
            jax.ShapeDtypeStruct((n, K), jnp.int32),
        ],
        interpret=INTERPRET,
    )(x2, w_gate)


def _ffn_body(meta_ref, xs_ref, wi_ref, wg_ref, wo_ref, sc_ref, ys_ref,
              wib, wgb, wob, sem0, sem1):
    t = pl.program_id(0)
    eid = meta_ref[0, t]
    rs = meta_ref[1, t]
    slot = meta_ref[2, t]
    peid = meta_ref[3, t]

    def fetch(e, s, ssem):
        for src, dst in ((wi_ref, wib), (wg_ref, wgb), (wo_ref, wob)):
            pltpu.make_async_copy(src.at[e], dst.at[s], ssem).start()

    def wait(e, s, ssem):
        for src, dst in ((wi_ref, wib), (wg_ref, wgb), (wo_ref, wob)):
            pltpu.make_async_copy(src.at[e], dst.at[s], ssem).wait()

    @pl.when(t == 0)
    def _():
        fetch(eid, 0, sem0)

    @pl.when(jnp.logical_and(rs == 1, peid >= 0))
    def _():
        @pl.when(slot == 0)
        def _():
            fetch(peid, 1, sem1)

        @pl.when(slot == 1)
        def _():
            fetch(peid, 0, sem0)

    @pl.when(rs == 1)
    def _():
        @pl.when(slot == 0)
        def _():
            wait(eid, 0, sem0)

        @pl.when(slot == 1)
        def _():
            wait(eid, 1, sem1)

    @pl.when(meta_ref[4, t] == 1)
    def _():
        cdt = jnp.bfloat16
        x = xs_ref[...].astype(cdt)
        up = lax.dot_general(x, wib[slot].astype(cdt), (((1,), (1,)), ((), ())),
                             preferred_element_type=jnp.float32)
        g = lax.dot_general(x, wgb[slot].astype(cdt), (((1,), (1,)), ((), ())),
                            preferred_element_type=jnp.float32)
        h = (g * jax.nn.sigmoid(g) * up).astype(cdt)
        y = lax.dot_general(h, wob[slot].astype(cdt), (((1,), (1,)), ((), ())),
                            preferred_element_type=jnp.float32)
        scale = sc_ref[0, 0, :].reshape(TM, 1)
        ys_ref[...] = y * scale


def _ffn(meta, xs, wi, wg, wo, scale3):
    s = xs.shape[0]
    nt = s // TM
    grid_spec = pltpu.PrefetchScalarGridSpec(
        num_scalar_prefetch=1,
        grid=(nt,),
        in_specs=[
            pl.BlockSpec((TM, DIM), lambda t, m: (t, 0)),
            pl.BlockSpec(memory_space=pltpu.MemorySpace.HBM),
            pl.BlockSpec(memory_space=pltpu.MemorySpace.HBM),
            pl.BlockSpec(memory_space=pltpu.MemorySpace.HBM),
            pl.BlockSpec((1, 1, TM), lambda t, m: (t, 0, 0)),
        ],
        out_specs=pl.BlockSpec((TM, DIM), lambda t, m: (t, 0)),
        scratch_shapes=[
            pltpu.VMEM((2, HID, DIM), jnp.float32),
            pltpu.VMEM((2, HID, DIM), jnp.float32),
            pltpu.VMEM((2, DIM, HID), jnp.float32),
            pltpu.SemaphoreType.DMA,
            pltpu.SemaphoreType.DMA,
        ],
    )
    return pl.pallas_call(
        _ffn_body,
        grid_spec=grid_spec,
        out_shape=jax.ShapeDtypeStruct((s, DIM), jnp.float32),
        interpret=INTERPRET,
    )(meta, xs, wi, wg, wo, scale3)


NP = 4096 * K
S = NP + E * TM
NSUB = 16
CH = NP // NSUB
NV = CH // 16
STRIPE = S // 32
GB = 64
CB = 32


RB = 32
NB = CH // RB


def _dispatch_body(idx_hbm, val_hbm, x_hbm,
                   xs_hbm, scale_hbm, pos_hbm, cnt_hbm,
                   e_v, v_v, posf_v, pos2_v, cnt_v, allcnt_v,
                   pos8_v, tok8_v, rows_v,
                   counts_sh, sem, sem2):
    cid = lax.axis_index("c")
    sid = lax.axis_index("s")
    lane = lax.broadcasted_iota(jnp.int32, (16,), 0)
    zero16 = jnp.zeros((16,), jnp.int32)
    pbase = sid * CH

    pltpu.sync_copy(idx_hbm.at[pl.ds(pbase, CH)], e_v)
    pltpu.sync_copy(val_hbm.at[pl.ds(pbase, CH)], v_v)

    def cnt_body(i, c):
        ev = e_v[pl.ds(i * 16, 16)]
        for e in range(E):
            incl = plsc.cumsum(jnp.where(ev == e, 1, 0))
            pc = jnp.broadcast_to(jnp.max(incl, axis=0), (16,))
            c = c + jnp.where(lane == e, pc, 0)
        return c
    cnt = lax.fori_loop(0, NV, cnt_body, zero16)
    cnt_v[...] = cnt
    pltpu.sync_copy(cnt_v, counts_sh.at[pl.ds(sid * 16, 16)])
    plsc.subcore_barrier()

    pltpu.sync_copy(counts_sh, allcnt_v)

    def acc_body(i, carry):
        tot, pre = carry
        row = allcnt_v[pl.ds(i * 16, 16)]
        return tot + row, pre + jnp.where(i < sid, row, 0)
    tot, pre = lax.fori_loop(0, NSUB, acc_body, (zero16, zero16))
    padded = ((tot + TM - 1) // TM) * TM
    starts = plsc.cumsum(padded) - padded
    base0 = starts + pre

    @pl.when(jnp.logical_and(cid == 0, sid == 0))
    def _():
        cnt_v[...] = tot
        pltpu.sync_copy(cnt_v, cnt_hbm)

    def pos_body(i, b):
        ev = e_v[pl.ds(i * 16, 16)]
        p = zero16
        for e in range(E):
            m = ev == e
            incl = plsc.cumsum(jnp.where(m, 1, 0))
            be = jnp.broadcast_to(jnp.max(jnp.where(lane == e, b, -1), axis=0),
                                  (16,))
            p = jnp.where(m, be + incl - 1, p)
            pc = jnp.broadcast_to(jnp.max(incl, axis=0), (16,))
            b = b + jnp.where(lane == e, pc, 0)
        posf_v[pl.ds(i * 16, 16)] = p
        pos2_v[i // 8, pl.ds((i % 8) * 16, 16)] = p
        tok = (pbase + i * 16 + lane) // K
        nl = RB // 16
        pos8_v[i // nl, pl.ds((i % nl) * 16, 16)] = p
        tok8_v[i // nl, pl.ds((i % nl) * 16, 16)] = tok
        return b
    lax.fori_loop(0, NV, pos_body, base0)

    @pl.when(cid == 0)
    def _():
        pltpu.sync_copy(posf_v, pos_hbm.at[pl.ds(pbase, CH)])

    co = pl.multiple_of(cid * (CH // 2), CH // 2)
    for j in range(CH // 256):
        jo = pl.multiple_of(co // 128 + j, 1)
        pltpu.sync_copy(v_v.at[pl.ds(co + j * 128, 128)],
                        scale_hbm.at[pos2_v.at[jo]])

    bo = cid * (NB // 2)
    nbh = NB // 2
    gcps = {}
    scps = {}
    for b in range(min(2, nbh)):
        gcps[b] = pltpu.async_copy(x_hbm.at[tok8_v.at[bo + b]],
                                   rows_v.at[b % 3], sem)
    for b in range(nbh):
        if b - 1 >= 0:
            scps[b - 1].wait()
        if b + 2 < nbh:
            gcps[b + 2] = pltpu.async_copy(x_hbm.at[tok8_v.at[bo + b + 2]],
                                           rows_v.at[(b + 2) % 3], sem)
        gcps[b].wait()
        scps[b] = pltpu.async_copy(rows_v.at[b % 3],
                                   xs_hbm.at[pos8_v.at[bo + b]], sem2)
    scps[nbh - 1].wait()


def _dispatch(idx, vals, x2):
    n = x2.shape[0]
    mesh = plsc.VectorSubcoreMesh(core_axis_name="c", subcore_axis_name="s")
    f = pl.kernel(
        _dispatch_body,
        out_type=[
            jax.ShapeDtypeStruct((S, DIM), jnp.float32),
            jax.ShapeDtypeStruct((S,), jnp.float32),
            jax.ShapeDtypeStruct((NP,), jnp.int32),
            jax.ShapeDtypeStruct((16,), jnp.int32),
        ],
        mesh=mesh,
        scratch_types=[
            pltpu.VMEM((CH,), jnp.int32),
            pltpu.VMEM((CH,), jnp.float32),
            pltpu.VMEM((CH,), jnp.int32),
            pltpu.VMEM((CH // 128, 128), jnp.int32),
            pltpu.VMEM((16,), jnp.int32),
            pltpu.VMEM((NSUB * 16,), jnp.int32),
            pltpu.VMEM((NB, RB), jnp.int32),
            pltpu.VMEM((NB, RB), jnp.int32),
            pltpu.VMEM((3, RB, DIM), jnp.float32),
            pltpu.VMEM_SHARED((NSUB * 16,), jnp.int32),
            pltpu.SemaphoreType.DMA,
            pltpu.SemaphoreType.DMA,
        ],
        compiler_params=pltpu.CompilerParams(needs_layout_passes=False),
    )
    return f(idx.reshape(NP), vals.reshape(NP), x2)


def _combine_body(ys_hbm, pos_hbm, out_hbm, posc_v, rows_v, out_v, sem, sem2):
    cid = lax.axis_index("c")
    sid = lax.axis_index("s")
    wid = sid * 2 + cid
    rbase = pl.multiple_of(wid * (NP // 32), NP // 32)
    nb = (NP // 32) // CB
    for b in range(nb):
        pltpu.sync_copy(pos_hbm.at[pl.ds(rbase + b * CB, CB)], posc_v.at[b])
    gcp = pltpu.async_copy(ys_hbm.at[posc_v.at[0]], rows_v.at[0], sem)
    ocp = None
    for b in range(nb):
        cur = b % 2
        if b + 1 < nb:
            ncp = pltpu.async_copy(ys_hbm.at[posc_v.at[b + 1]],
                                   rows_v.at[1 - cur], sem)
        gcp.wait()
        if ocp is not None:
            ocp.wait()

        def add_body(j, _):
            for t in range(CB // K):
                a = rows_v[cur, K * t, pl.ds(j * 16, 16)]
                c = rows_v[cur, K * t + 1, pl.ds(j * 16, 16)]
                out_v[cur, t, pl.ds(j * 16, 16)] = a + c
            return 0
        lax.fori_loop(0, DIM // 16, add_body, 0)
        obase = pl.multiple_of((rbase + b * CB) // K, CB // K)
        ocp = pltpu.async_copy(out_v.at[cur], out_hbm.at[pl.ds(obase, CB // K)],
                               sem2)
        if b + 1 < nb:
            gcp = ncp
    ocp.wait()


def _combine(ys, pos, n):
    mesh = plsc.VectorSubcoreMesh(core_axis_name="c", subcore_axis_name="s")
    f = pl.kernel(
        _combine_body,
        out_type=jax.ShapeDtypeStruct((n, DIM), jnp.float32),
        mesh=mesh,
        scratch_types=[
            pltpu.VMEM(((NP // 32) // CB, CB), jnp.int32),
            pltpu.VMEM((2, CB, DIM), jnp.float32),
            pltpu.VMEM((2, CB // K, DIM), jnp.float32),
            pltpu.SemaphoreType.DMA,
            pltpu.SemaphoreType.DMA,
        ],
        compiler_params=pltpu.CompilerParams(needs_layout_passes=False),
    )
    return f(ys, pos)


def kernel(x, w_gate, wi, wg, wo):
    b, t, _ = x.shape
    n = b * t
    nt = S // TM

    x2 = x.reshape(n, DIM)
    vals, idx = _router(x2, w_gate)

    xs, scale, pos, cnt16 = _dispatch(idx, vals, x2)

    counts = cnt16[:E]
    padded = ((counts + TM - 1) // TM) * TM
    ends = jnp.cumsum(padded).astype(jnp.int32)
    idxs = jnp.arange(nt, dtype=jnp.int32)
    gid = jnp.minimum(
        jnp.sum((idxs[:, None] * TM) >= ends[None, :], axis=1),
        E - 1).astype(jnp.int32)
    rs = jnp.concatenate([jnp.ones((1,), jnp.int32),
                          (gid[1:] != gid[:-1]).astype(jnp.int32)])
    run_id = jnp.cumsum(rs).astype(jnp.int32) - 1
    slot = run_id % 2
    ns = jnp.where(rs == 1, idxs, 2 * nt)
    suf = lax.associative_scan(jnp.minimum, ns[::-1])[::-1]
    nxt = jnp.concatenate([suf[1:], jnp.full((1,), 2 * nt, jnp.int32)])
    peid = jnp.where((rs == 1) & (nxt < nt), gid[jnp.clip(nxt, 0, nt - 1)], -1)
    act = (idxs * TM < ends[E - 1]).astype(jnp.int32)
    meta = jnp.stack([gid, rs, slot, peid.astype(jnp.int32), act])

    ys = _ffn(meta, xs, wi, wg, wo, scale.reshape(nt, 1, TM))

    out = _combine(ys, pos, n)
    return out.reshape(b, t, DIM)
